# Initial kernel scaffold; baseline (speedup 1.0000x reference)
import jax
import jax.numpy as jnp
from jax.experimental import pallas as pl


def kernel(z, table):
    raise NotImplementedError("write your pallas kernel here")


# speedup vs baseline: 1.5089x; 1.5089x over previous
import functools

import jax
import jax.numpy as jnp
from jax import lax
from jax.experimental import pallas as pl
from jax.experimental.pallas import tpu as pltpu
from jax.experimental.pallas import tpu_sc as plsc

B = 100000
D = 128
NC = 2
NS = 16
NW = NC * NS
CHUNK = 512
GPC = CHUNK // 128
NCHUNK = (B + CHUNK - 1) // CHUNK
LAST_BASE = B - CHUNK


def _body(z_hbm, table_hbm, out_hbm, idx_v, rows_v, sem):
    wid = lax.axis_index("s") * NC + lax.axis_index("c")
    nloc = (NCHUNK - wid + NW - 1) // NW

    def chunk_step(i, carry):
        cid = wid + i * NW
        base = lax.min(cid * CHUNK, LAST_BASE)
        base = pl.multiple_of(base, 8)
        pltpu.sync_copy(z_hbm.at[pl.ds(base, CHUNK)], idx_v)
        descs = []
        for j in range(GPC):
            descs.append(
                pltpu.async_copy(
                    table_hbm.at[idx_v.at[pl.ds(j * 128, 128)]],
                    rows_v.at[pl.ds(j * 128, 128)],
                    sem,
                )
            )
        for d in descs:
            d.wait()
        pltpu.sync_copy(rows_v, out_hbm.at[pl.ds(base, CHUNK)])
        return carry

    lax.fori_loop(0, nloc, chunk_step, 0)


@jax.jit
def kernel(z, table):
    z = z.astype(jnp.int32)
    mesh = plsc.VectorSubcoreMesh(core_axis_name="c", subcore_axis_name="s")
    f = pl.kernel(
        _body,
        out_type=jax.ShapeDtypeStruct((B, D), jnp.float32),
        mesh=mesh,
        scratch_types=[
            pltpu.VMEM((CHUNK,), jnp.int32),
            pltpu.VMEM((CHUNK, D), jnp.float32),
            pltpu.SemaphoreType.DMA,
        ],
    )
    return f(z, table)


# speedup vs baseline: 1.5223x; 1.0089x over previous
import jax
import jax.numpy as jnp
from jax import lax
from jax.experimental import pallas as pl
from jax.experimental.pallas import tpu as pltpu
from jax.experimental.pallas import tpu_sc as plsc

B = 100000
D = 128
NC = 2
NS = 16
NW = NC * NS
CHUNK = 384
GPC = CHUNK // 128
NCHUNK = (B + CHUNK - 1) // CHUNK
LAST_BASE = B - CHUNK
MAX_LOC = (NCHUNK + NW - 1) // NW


def _body(z_hbm, table_hbm, out_hbm, idx0, idx1, rows0, rows1, g0, g1, w0, w1):
    wid = lax.axis_index("s") * NC + lax.axis_index("c")
    nloc = (NCHUNK - wid + NW - 1) // NW
    idxs = (idx0, idx1)
    rows = (rows0, rows1)
    gsems = (g0, g1)
    wsems = (w0, w1)

    def base_of(i):
        cid = wid + i * NW
        return pl.multiple_of(lax.min(cid * CHUNK, LAST_BASE), 8)

    def fire_gather(i, b):
        base = base_of(i)
        pltpu.sync_copy(z_hbm.at[pl.ds(base, CHUNK)], idxs[b])
        for j in range(GPC):
            pltpu.async_copy(
                table_hbm.at[idxs[b].at[pl.ds(j * 128, 128)]],
                rows[b].at[pl.ds(j * 128, 128)],
                gsems[b],
            )

    def wait_gather(b):
        for j in range(GPC):
            pltpu.make_async_copy(
                table_hbm.at[idxs[b].at[pl.ds(j * 128, 128)]],
                rows[b].at[pl.ds(j * 128, 128)],
                gsems[b],
            ).wait()

    def fire_write(i, b):
        pltpu.async_copy(rows[b], out_hbm.at[pl.ds(base_of(i), CHUNK)], wsems[b])

    def wait_write(i, b):
        pltpu.make_async_copy(
            rows[b], out_hbm.at[pl.ds(base_of(i), CHUNK)], wsems[b]
        ).wait()

    fire_gather(0, 0)

    def step(i, b):
        @pl.when(i < nloc)
        def _():
            wait_gather(b)
            fire_write(i, b)

        @pl.when(i + 1 < nloc)
        def _():
            @pl.when(i >= 1)
            def _():
                wait_write(i - 1, 1 - b)

            fire_gather(i + 1, 1 - b)

    def loop_body(k, carry):
        step(2 * k, 0)
        step(2 * k + 1, 1)
        return carry

    lax.fori_loop(0, (MAX_LOC + 1) // 2, loop_body, 0)

    even = (nloc % 2) == 0

    @pl.when(even)
    def _():
        wait_write(nloc - 2, 0)
        wait_write(nloc - 1, 1)

    @pl.when(jnp.logical_not(even))
    def _():
        wait_write(nloc - 2, 1)
        wait_write(nloc - 1, 0)


@jax.jit
def kernel(z, table):
    z = z.astype(jnp.int32)
    mesh = plsc.VectorSubcoreMesh(core_axis_name="c", subcore_axis_name="s")
    f = pl.kernel(
        _body,
        out_type=jax.ShapeDtypeStruct((B, D), jnp.float32),
        mesh=mesh,
        scratch_types=[
            pltpu.VMEM((CHUNK,), jnp.int32),
            pltpu.VMEM((CHUNK,), jnp.int32),
            pltpu.VMEM((CHUNK, D), jnp.float32),
            pltpu.VMEM((CHUNK, D), jnp.float32),
            pltpu.SemaphoreType.DMA,
            pltpu.SemaphoreType.DMA,
            pltpu.SemaphoreType.DMA,
            pltpu.SemaphoreType.DMA,
        ],
    )
    return f(z, table)
